# Initial kernel scaffold; baseline (speedup 1.0000x reference)
import functools

import jax
import jax.numpy as jnp
from jax import lax
from jax.experimental import pallas as pl
from jax.experimental.pallas import tpu as pltpu
from jax.experimental.pallas import tpu_sc as plsc

N = 10000
E = 320000
NFEAT = 128
NHID = 128
NCLASS = 64

NC = 2
NS = 16
NW = NC * NS
CHUNK = 128
CHUNKS_PER_TILE = -(-E // (NW * CHUNK))
E_PAD = NW * CHUNK * CHUNKS_PER_TILE
NPAD = 10240
ROWS_PER_TILE = NPAD // NS
DUMMY_DST = N + 1


def _zero_fill(ref, nrows, ncols):
    z = jnp.zeros((16,), jnp.float32)
    for i in range(nrows):
        for j in range(ncols // 16):
            ref[i, pl.ds(j * 16, 16)] = z


def _sc_aggregate(feat_dim, with_degree):
    mesh = plsc.VectorSubcoreMesh(core_axis_name="c", subcore_axis_name="s")
    out_type = [jax.ShapeDtypeStruct((NC, NPAD, feat_dim), jnp.float32)]
    scratch = [
        pltpu.VMEM_SHARED((NPAD, feat_dim), jnp.float32),
        pltpu.VMEM((CHUNKS_PER_TILE, CHUNK), jnp.int32),
        pltpu.VMEM((CHUNKS_PER_TILE, CHUNK), jnp.int32),
        pltpu.VMEM((CHUNK, feat_dim), jnp.float32),
        pltpu.VMEM((16, feat_dim), jnp.float32),
        pltpu.SemaphoreType.DMA,
    ]
    if with_degree:
        out_type.append(jax.ShapeDtypeStruct((NC, NPAD, 16), jnp.float32))
        scratch += [
            pltpu.VMEM_SHARED((NPAD, 16), jnp.float32),
            pltpu.VMEM((CHUNK, 16), jnp.float32),
            pltpu.VMEM((16, 16), jnp.float32),
        ]

    @functools.partial(
        pl.kernel,
        out_type=tuple(out_type),
        mesh=mesh,
        scratch_types=tuple(scratch),
    )
    def k(table_hbm, src3_hbm, dst3_hbm, *refs):
        if with_degree:
            (out_hbm, deg_hbm, acc_sh, src_v, dst_v, rows_v, zbuf, sem,
             deg_sh, ones_v, zbufd) = refs
        else:
            out_hbm, acc_sh, src_v, dst_v, rows_v, zbuf, sem = refs

        cid = lax.axis_index("c")
        sid = lax.axis_index("s")
        wid = sid * NC + cid

        _zero_fill(zbuf, 16, feat_dim)
        if with_degree:
            _zero_fill(zbufd, 16, 16)
            one = jnp.ones((16,), jnp.float32)

            def fill_ones(i, _):
                ones_v[i, :] = one
                return 0

            lax.fori_loop(0, CHUNK, fill_ones, 0)

        row0 = sid * ROWS_PER_TILE

        def zero_body(i, _):
            pltpu.sync_copy(zbuf, acc_sh.at[pl.ds(row0 + i * 16, 16)])
            if with_degree:
                pltpu.sync_copy(zbufd, deg_sh.at[pl.ds(row0 + i * 16, 16)])
            return 0

        lax.fori_loop(0, ROWS_PER_TILE // 16, zero_body, 0)

        pltpu.sync_copy(src3_hbm.at[wid], src_v)
        pltpu.sync_copy(dst3_hbm.at[wid], dst_v)

        plsc.subcore_barrier()

        def chunk_body(c, _):
            pltpu.async_copy(table_hbm.at[src_v.at[c]], rows_v, sem).wait()
            pltpu.sync_copy(rows_v, acc_sh.at[dst_v.at[c]], add=True)
            if with_degree:
                pltpu.sync_copy(ones_v, deg_sh.at[dst_v.at[c]], add=True)
            return 0

        lax.fori_loop(0, CHUNKS_PER_TILE, chunk_body, 0)

        plsc.subcore_barrier()

        pltpu.sync_copy(
            acc_sh.at[pl.ds(row0, ROWS_PER_TILE)],
            out_hbm.at[cid, pl.ds(row0, ROWS_PER_TILE)],
        )
        if with_degree:
            pltpu.sync_copy(
                deg_sh.at[pl.ds(row0, ROWS_PER_TILE)],
                deg_hbm.at[cid, pl.ds(row0, ROWS_PER_TILE)],
            )

    return k


_sc_agg_l1 = _sc_aggregate(NFEAT, with_degree=True)
_sc_agg_l2 = _sc_aggregate(NCLASS, with_degree=False)

BR = 512


def _tc_layer1_body(agg_ref, deg_ref, x_ref, w1l_ref, b1_ref, w1r_ref,
                    w2l_ref, w2r_ref, hl_ref, hr_ref):
    agg = agg_ref[0] + agg_ref[1]
    deg = deg_ref[0, :, :1] + deg_ref[1, :, :1]
    inv = 1.0 / jnp.maximum(deg, 1.0)
    mean = agg * inv
    dn = (((1,), (1,)), ((), ()))
    h = (lax.dot_general(mean, w1l_ref[...], dn,
                         preferred_element_type=jnp.float32)
         + b1_ref[...]
         + lax.dot_general(x_ref[...], w1r_ref[...], dn,
                           preferred_element_type=jnp.float32))
    hl_ref[...] = lax.dot_general(h, w2l_ref[...], dn,
                                  preferred_element_type=jnp.float32)
    hr_ref[...] = lax.dot_general(h, w2r_ref[...], dn,
                                  preferred_element_type=jnp.float32)


def _tc_layer2_body(agg_ref, deg_ref, hr_ref, b2_ref, out_ref):
    agg = agg_ref[0] + agg_ref[1]
    deg = deg_ref[0, :, :1] + deg_ref[1, :, :1]
    inv = 1.0 / jnp.maximum(deg, 1.0)
    z = agg * inv + b2_ref[...] + hr_ref[...]
    m = jnp.max(z, axis=1, keepdims=True)
    lse = m + jnp.log(jnp.sum(jnp.exp(z - m), axis=1, keepdims=True))
    out_ref[...] = z - lse


def _tc_layer1(agg1, deg, x_pad, W1l, b1, W1r, W2l, W2r):
    grid = (NPAD // BR,)
    return pl.pallas_call(
        _tc_layer1_body,
        grid=grid,
        in_specs=[
            pl.BlockSpec((NC, BR, NHID), lambda r: (0, r, 0)),
            pl.BlockSpec((NC, BR, 16), lambda r: (0, r, 0)),
            pl.BlockSpec((BR, NFEAT), lambda r: (r, 0)),
            pl.BlockSpec((NHID, NFEAT), lambda r: (0, 0)),
            pl.BlockSpec((1, NHID), lambda r: (0, 0)),
            pl.BlockSpec((NHID, NFEAT), lambda r: (0, 0)),
            pl.BlockSpec((NCLASS, NHID), lambda r: (0, 0)),
            pl.BlockSpec((NCLASS, NHID), lambda r: (0, 0)),
        ],
        out_specs=[
            pl.BlockSpec((BR, NCLASS), lambda r: (r, 0)),
            pl.BlockSpec((BR, NCLASS), lambda r: (r, 0)),
        ],
        out_shape=[
            jax.ShapeDtypeStruct((NPAD, NCLASS), jnp.float32),
            jax.ShapeDtypeStruct((NPAD, NCLASS), jnp.float32),
        ],
    )(agg1, deg, x_pad, W1l, b1, W1r, W2l, W2r)


def _tc_layer2(agg2, deg, hr, b2):
    grid = (NPAD // BR,)
    return pl.pallas_call(
        _tc_layer2_body,
        grid=grid,
        in_specs=[
            pl.BlockSpec((NC, BR, NCLASS), lambda r: (0, r, 0)),
            pl.BlockSpec((NC, BR, 16), lambda r: (0, r, 0)),
            pl.BlockSpec((BR, NCLASS), lambda r: (r, 0)),
            pl.BlockSpec((1, NCLASS), lambda r: (0, 0)),
        ],
        out_specs=pl.BlockSpec((BR, NCLASS), lambda r: (r, 0)),
        out_shape=jax.ShapeDtypeStruct((NPAD, NCLASS), jnp.float32),
    )(agg2, deg, hr, b2)


@jax.jit
def kernel(x, edge_index, W1l, b1, W1r, W2l, b2, W2r):
    src = edge_index[0]
    dst = edge_index[1]
    pad = E_PAD - E
    srcp = jnp.concatenate([src, jnp.zeros((pad,), jnp.int32)])
    dstp = jnp.concatenate([dst, jnp.full((pad,), DUMMY_DST, jnp.int32)])
    src3 = srcp.reshape(NW, CHUNKS_PER_TILE, CHUNK)
    dst3 = dstp.reshape(NW, CHUNKS_PER_TILE, CHUNK)

    agg1, deg = _sc_agg_l1(x, src3, dst3)

    x_pad = jnp.pad(x, ((0, NPAD - N), (0, 0)))
    hl, hr = _tc_layer1(agg1, deg, x_pad, W1l, b1.reshape(1, NHID), W1r,
                        W2l, W2r)

    (agg2,) = _sc_agg_l2(hl, src3, dst3)

    out = _tc_layer2(agg2, deg, hr, b2.reshape(1, NCLASS))
    return out[:N]


# speedup vs baseline: 6.5713x; 6.5713x over previous
import functools

import jax
import jax.numpy as jnp
from jax import lax
from jax.experimental import pallas as pl
from jax.experimental.pallas import tpu as pltpu
from jax.experimental.pallas import tpu_sc as plsc

N = 10000
E = 320000
NFEAT = 128
NHID = 128
NCLASS = 64

NC = 2
NS = 16
CHUNK = 128
CHUNKS_PER_TILE = -(-E // (NS * CHUNK))
E_PAD = NS * CHUNK * CHUNKS_PER_TILE
NPAD = 10240
ROWS_PER_TILE = NPAD // NS
DUMMY_DST = N + 1


def _zero_fill(ref, nrows, ncols):
    z = jnp.zeros((16,), jnp.float32)
    for i in range(nrows):
        for j in range(ncols // 16):
            ref[i, pl.ds(j * 16, 16)] = z


def _sc_aggregate(cw, with_degree):
    mesh = plsc.VectorSubcoreMesh(core_axis_name="c", subcore_axis_name="s")
    out_type = [jax.ShapeDtypeStruct((NC, NPAD, cw), jnp.float32)]
    scratch = [
        pltpu.VMEM_SHARED((NPAD, cw), jnp.float32),
        pltpu.VMEM((CHUNKS_PER_TILE, CHUNK), jnp.int32),
        pltpu.VMEM((CHUNKS_PER_TILE, CHUNK), jnp.int32),
        pltpu.VMEM((CHUNK, cw), jnp.float32),
        pltpu.VMEM((16, cw), jnp.float32),
        pltpu.SemaphoreType.DMA,
    ]
    if with_degree:
        out_type.append(jax.ShapeDtypeStruct((NPAD, 16), jnp.float32))
        scratch += [
            pltpu.VMEM_SHARED((NPAD, 16), jnp.float32),
            pltpu.VMEM((CHUNK, 16), jnp.float32),
            pltpu.VMEM((16, 16), jnp.float32),
        ]

    @functools.partial(
        pl.kernel,
        out_type=tuple(out_type),
        mesh=mesh,
        scratch_types=tuple(scratch),
        compiler_params=pltpu.CompilerParams(use_tc_tiling_on_sc=False),
    )
    def k(table_hbm, src4_hbm, dst3_hbm, *refs):
        if with_degree:
            (out_hbm, deg_hbm, acc_sh, src_v, dst_v, rows_v, zbuf, sem,
             deg_sh, ones_v, zbufd) = refs
        else:
            out_hbm, acc_sh, src_v, dst_v, rows_v, zbuf, sem = refs

        cid = lax.axis_index("c")
        sid = lax.axis_index("s")

        _zero_fill(zbuf, 16, cw)
        if with_degree:
            _zero_fill(zbufd, 16, 16)
            one = jnp.ones((16,), jnp.float32)

            def fill_ones(i, _):
                ones_v[i, :] = one
                return 0

            lax.fori_loop(0, CHUNK, fill_ones, 0)

        row0 = sid * ROWS_PER_TILE

        def zero_body(i, _):
            pltpu.sync_copy(zbuf, acc_sh.at[pl.ds(row0 + i * 16, 16)])
            if with_degree:
                pltpu.sync_copy(zbufd, deg_sh.at[pl.ds(row0 + i * 16, 16)])
            return 0

        lax.fori_loop(0, ROWS_PER_TILE // 16, zero_body, 0)

        pltpu.sync_copy(src4_hbm.at[cid, sid], src_v)
        pltpu.sync_copy(dst3_hbm.at[sid], dst_v)

        plsc.subcore_barrier()

        if with_degree:
            def chunk_body_deg(c, _):
                pltpu.async_copy(table_hbm.at[src_v.at[c]], rows_v, sem).wait()
                pltpu.sync_copy(rows_v, acc_sh.at[dst_v.at[c]], add=True)
                pltpu.sync_copy(ones_v, deg_sh.at[dst_v.at[c]], add=True)
                return 0

            def chunk_body_nodeg(c, _):
                pltpu.async_copy(table_hbm.at[src_v.at[c]], rows_v, sem).wait()
                pltpu.sync_copy(rows_v, acc_sh.at[dst_v.at[c]], add=True)
                return 0

            @pl.when(cid == 0)
            def _():
                lax.fori_loop(0, CHUNKS_PER_TILE, chunk_body_deg, 0)

            @pl.when(cid != 0)
            def _():
                lax.fori_loop(0, CHUNKS_PER_TILE, chunk_body_nodeg, 0)
        else:
            def chunk_body(c, _):
                pltpu.async_copy(table_hbm.at[src_v.at[c]], rows_v, sem).wait()
                pltpu.sync_copy(rows_v, acc_sh.at[dst_v.at[c]], add=True)
                return 0

            lax.fori_loop(0, CHUNKS_PER_TILE, chunk_body, 0)

        plsc.subcore_barrier()

        pltpu.sync_copy(
            acc_sh.at[pl.ds(row0, ROWS_PER_TILE)],
            out_hbm.at[cid, pl.ds(row0, ROWS_PER_TILE)],
        )
        if with_degree:
            @pl.when(cid == 0)
            def _():
                pltpu.sync_copy(
                    deg_sh.at[pl.ds(row0, ROWS_PER_TILE)],
                    deg_hbm.at[pl.ds(row0, ROWS_PER_TILE)],
                )

    return k


_sc_agg_l1 = _sc_aggregate(NFEAT // NC, with_degree=True)
_sc_agg_l2 = _sc_aggregate(NCLASS // NC, with_degree=False)

BR = 512
CW1 = NFEAT // NC
CW2 = NCLASS // NC


def _tc_layer1_body(agg_ref, deg_ref, x_ref, w1l_ref, b1_ref, w1r_ref,
                    w2l_ref, w2r_ref, hl_ref, hr_ref):
    agg = jnp.concatenate([agg_ref[0], agg_ref[1]], axis=1)
    deg = deg_ref[:, :1]
    inv = 1.0 / jnp.maximum(deg, 1.0)
    mean = agg * inv
    dn = (((1,), (1,)), ((), ()))
    h = (lax.dot_general(mean, w1l_ref[...], dn,
                         preferred_element_type=jnp.float32)
         + b1_ref[...]
         + lax.dot_general(x_ref[...], w1r_ref[...], dn,
                           preferred_element_type=jnp.float32))
    hl = lax.dot_general(h, w2l_ref[...], dn,
                         preferred_element_type=jnp.float32)
    hl_ref[0] = hl[:, :CW2]
    hl_ref[1] = hl[:, CW2:]
    hr_ref[...] = lax.dot_general(h, w2r_ref[...], dn,
                                  preferred_element_type=jnp.float32)


def _tc_layer2_body(agg_ref, deg_ref, hr_ref, b2_ref, out_ref):
    agg = jnp.concatenate([agg_ref[0], agg_ref[1]], axis=1)
    deg = deg_ref[:, :1]
    inv = 1.0 / jnp.maximum(deg, 1.0)
    z = agg * inv + b2_ref[...] + hr_ref[...]
    m = jnp.max(z, axis=1, keepdims=True)
    lse = m + jnp.log(jnp.sum(jnp.exp(z - m), axis=1, keepdims=True))
    out_ref[...] = z - lse


def _tc_layer1(agg1, deg, x_pad, W1l, b1, W1r, W2l, W2r):
    grid = (NPAD // BR,)
    return pl.pallas_call(
        _tc_layer1_body,
        grid=grid,
        in_specs=[
            pl.BlockSpec((NC, BR, CW1), lambda r: (0, r, 0)),
            pl.BlockSpec((BR, 16), lambda r: (r, 0)),
            pl.BlockSpec((BR, NFEAT), lambda r: (r, 0)),
            pl.BlockSpec((NHID, NFEAT), lambda r: (0, 0)),
            pl.BlockSpec((1, NHID), lambda r: (0, 0)),
            pl.BlockSpec((NHID, NFEAT), lambda r: (0, 0)),
            pl.BlockSpec((NCLASS, NHID), lambda r: (0, 0)),
            pl.BlockSpec((NCLASS, NHID), lambda r: (0, 0)),
        ],
        out_specs=[
            pl.BlockSpec((NC, BR, CW2), lambda r: (0, r, 0)),
            pl.BlockSpec((BR, NCLASS), lambda r: (r, 0)),
        ],
        out_shape=[
            jax.ShapeDtypeStruct((NC, NPAD, CW2), jnp.float32),
            jax.ShapeDtypeStruct((NPAD, NCLASS), jnp.float32),
        ],
    )(agg1, deg, x_pad, W1l, b1, W1r, W2l, W2r)


def _tc_layer2(agg2, deg, hr, b2):
    grid = (NPAD // BR,)
    return pl.pallas_call(
        _tc_layer2_body,
        grid=grid,
        in_specs=[
            pl.BlockSpec((NC, BR, CW2), lambda r: (0, r, 0)),
            pl.BlockSpec((BR, 16), lambda r: (r, 0)),
            pl.BlockSpec((BR, NCLASS), lambda r: (r, 0)),
            pl.BlockSpec((1, NCLASS), lambda r: (0, 0)),
        ],
        out_specs=pl.BlockSpec((BR, NCLASS), lambda r: (r, 0)),
        out_shape=jax.ShapeDtypeStruct((NPAD, NCLASS), jnp.float32),
    )(agg2, deg, hr, b2)


@jax.jit
def kernel(x, edge_index, W1l, b1, W1r, W2l, b2, W2r):
    src = edge_index[0]
    dst = edge_index[1]
    pad = E_PAD - E
    srcp = jnp.concatenate([src, jnp.zeros((pad,), jnp.int32)])
    dstp = jnp.concatenate([dst, jnp.full((pad,), DUMMY_DST, jnp.int32)])
    src3 = srcp.reshape(NS, CHUNKS_PER_TILE, CHUNK)
    src4 = jnp.stack([src3, src3 + NPAD])
    dst3 = dstp.reshape(NS, CHUNKS_PER_TILE, CHUNK)

    x_pad = jnp.pad(x, ((0, NPAD - N), (0, 0)))
    xcat = jnp.concatenate([x_pad[:, :CW1], x_pad[:, CW1:]], axis=0)

    agg1, deg = _sc_agg_l1(xcat, src4, dst3)

    hl, hr = _tc_layer1(agg1, deg, x_pad, W1l, b1.reshape(1, NHID), W1r,
                        W2l, W2r)

    (agg2,) = _sc_agg_l2(hl.reshape(NC * NPAD, CW2), src4, dst3)

    out = _tc_layer2(agg2, deg, hr, b2.reshape(1, NCLASS))
    return out[:N]


# speedup vs baseline: 8.2634x; 1.2575x over previous
import functools

import jax
import jax.numpy as jnp
from jax import lax
from jax.experimental import pallas as pl
from jax.experimental.pallas import tpu as pltpu
from jax.experimental.pallas import tpu_sc as plsc

N = 10000
E = 320000
NFEAT = 128
NHID = 128
NCLASS = 64

NC = 2
NS = 16
CHUNK = 128
CHUNKS_PER_TILE = 158
E_PAD = NS * CHUNK * CHUNKS_PER_TILE
NPAIR = CHUNKS_PER_TILE // 2
HALF_CHUNKS = CHUNKS_PER_TILE // 2
NPAD = 10240
ROWS_PER_TILE = NPAD // NS
ZROWS = 64
DUMMY_DST = N + 1


def _sc_aggregate(cw, with_degree):
    mesh = plsc.VectorSubcoreMesh(core_axis_name="c", subcore_axis_name="s")
    out_type = [jax.ShapeDtypeStruct((NC, NPAD, cw), jnp.float32)]
    scratch = [
        pltpu.VMEM_SHARED((NPAD, cw), jnp.float32),
        pltpu.VMEM((CHUNKS_PER_TILE, CHUNK), jnp.int32),
        pltpu.VMEM((CHUNKS_PER_TILE, CHUNK), jnp.int32),
        pltpu.VMEM((CHUNK, cw), jnp.float32),
        pltpu.VMEM((CHUNK, cw), jnp.float32),
        pltpu.VMEM((ZROWS, cw), jnp.float32),
        pltpu.SemaphoreType.DMA,
        pltpu.SemaphoreType.DMA,
        pltpu.SemaphoreType.DMA,
    ]
    if with_degree:
        out_type.append(jax.ShapeDtypeStruct((NC, NPAD, 16), jnp.float32))
        scratch += [
            pltpu.VMEM_SHARED((NPAD, 16), jnp.float32),
            pltpu.VMEM((CHUNK, 16), jnp.float32),
            pltpu.VMEM((ZROWS, 16), jnp.float32),
        ]

    @functools.partial(
        pl.kernel,
        out_type=tuple(out_type),
        mesh=mesh,
        scratch_types=tuple(scratch),
        compiler_params=pltpu.CompilerParams(use_tc_tiling_on_sc=False),
    )
    def k(table_hbm, src4_hbm, dst3_hbm, *refs):
        if with_degree:
            (out_hbm, deg_hbm, acc_sh, src_v, dst_v, rows0, rows1, zbuf,
             gsem0, gsem1, zsem, deg_sh, ones_v, zbufd) = refs
        else:
            (out_hbm, acc_sh, src_v, dst_v, rows0, rows1, zbuf,
             gsem0, gsem1, zsem) = refs

        cid = lax.axis_index("c")
        sid = lax.axis_index("s")

        z = jnp.zeros((16,), jnp.float32)

        def fill_z(i, _):
            for j in range(cw // 16):
                zbuf[i, pl.ds(j * 16, 16)] = z
            if with_degree:
                zbufd[i, :] = z
            return 0

        lax.fori_loop(0, ZROWS, fill_z, 0)

        if with_degree:
            one = jnp.ones((16,), jnp.float32)

            def fill_ones(i, _):
                ones_v[i, :] = one
                return 0

            lax.fori_loop(0, CHUNK, fill_ones, 0)

        row0 = sid * ROWS_PER_TILE
        nz = ROWS_PER_TILE // ZROWS

        def zero_body(i, _):
            pltpu.async_copy(zbuf, acc_sh.at[pl.ds(row0 + i * ZROWS, ZROWS)],
                             zsem)
            if with_degree:
                pltpu.async_copy(
                    zbufd, deg_sh.at[pl.ds(row0 + i * ZROWS, ZROWS)], zsem)
            return 0

        lax.fori_loop(0, nz, zero_body, 0)

        pltpu.sync_copy(src4_hbm.at[cid, sid], src_v)
        pltpu.sync_copy(dst3_hbm.at[sid], dst_v)

        def zero_drain(i, _):
            pltpu.make_async_copy(
                zbuf, acc_sh.at[pl.ds(row0, ZROWS)], zsem).wait()
            if with_degree:
                pltpu.make_async_copy(
                    zbufd, deg_sh.at[pl.ds(row0, ZROWS)], zsem).wait()
            return 0

        lax.fori_loop(0, nz, zero_drain, 0)

        plsc.subcore_barrier()

        def fire(c, buf, sem):
            pltpu.async_copy(table_hbm.at[src_v.at[c]], buf, sem)

        def wait_g(buf, sem):
            pltpu.make_async_copy(table_hbm.at[src_v.at[0]], buf, sem).wait()

        def scat(c, buf):
            pltpu.sync_copy(buf, acc_sh.at[dst_v.at[c]], add=True)
            if with_degree:
                do = jnp.logical_or(
                    jnp.logical_and(cid == 0, c < HALF_CHUNKS),
                    jnp.logical_and(cid != 0, c >= HALF_CHUNKS))

                @pl.when(do)
                def _():
                    pltpu.sync_copy(ones_v, deg_sh.at[dst_v.at[c]], add=True)

        fire(0, rows0, gsem0)

        def pair_body(g, _):
            c0 = 2 * g
            c1 = c0 + 1
            fire(c1, rows1, gsem1)
            wait_g(rows0, gsem0)
            scat(c0, rows0)

            @pl.when(g + 1 < NPAIR)
            def _():
                fire(c0 + 2, rows0, gsem0)

            wait_g(rows1, gsem1)
            scat(c1, rows1)
            return 0

        lax.fori_loop(0, NPAIR, pair_body, 0)

        plsc.subcore_barrier()

        pltpu.async_copy(
            acc_sh.at[pl.ds(row0, ROWS_PER_TILE)],
            out_hbm.at[cid, pl.ds(row0, ROWS_PER_TILE)],
            zsem)
        if with_degree:
            pltpu.async_copy(
                deg_sh.at[pl.ds(row0, ROWS_PER_TILE)],
                deg_hbm.at[cid, pl.ds(row0, ROWS_PER_TILE)],
                zsem)
            pltpu.make_async_copy(
                deg_sh.at[pl.ds(row0, ROWS_PER_TILE)],
                deg_hbm.at[cid, pl.ds(row0, ROWS_PER_TILE)],
                zsem).wait()
        pltpu.make_async_copy(
            acc_sh.at[pl.ds(row0, ROWS_PER_TILE)],
            out_hbm.at[cid, pl.ds(row0, ROWS_PER_TILE)],
            zsem).wait()

    return k


_sc_agg_l1 = _sc_aggregate(NFEAT // NC, with_degree=True)
_sc_agg_l2 = _sc_aggregate(NCLASS // NC, with_degree=False)

BR = 512
CW1 = NFEAT // NC
CW2 = NCLASS // NC


def _tc_layer1_body(agg_ref, deg_ref, x_ref, w1l_ref, b1_ref, w1r_ref,
                    w2l_ref, w2r_ref, hl_ref, hr_ref):
    agg = jnp.concatenate([agg_ref[0], agg_ref[1]], axis=1)
    deg = deg_ref[0, :, :1] + deg_ref[1, :, :1]
    inv = 1.0 / jnp.maximum(deg, 1.0)
    mean = agg * inv
    dn = (((1,), (1,)), ((), ()))
    h = (lax.dot_general(mean, w1l_ref[...], dn,
                         preferred_element_type=jnp.float32)
         + b1_ref[...]
         + lax.dot_general(x_ref[...], w1r_ref[...], dn,
                           preferred_element_type=jnp.float32))
    hl = lax.dot_general(h, w2l_ref[...], dn,
                         preferred_element_type=jnp.float32)
    hl_ref[0] = hl[:, :CW2]
    hl_ref[1] = hl[:, CW2:]
    hr_ref[...] = lax.dot_general(h, w2r_ref[...], dn,
                                  preferred_element_type=jnp.float32)


def _tc_layer2_body(agg_ref, deg_ref, hr_ref, b2_ref, out_ref):
    agg = jnp.concatenate([agg_ref[0], agg_ref[1]], axis=1)
    deg = deg_ref[0, :, :1] + deg_ref[1, :, :1]
    inv = 1.0 / jnp.maximum(deg, 1.0)
    z = agg * inv + b2_ref[...] + hr_ref[...]
    m = jnp.max(z, axis=1, keepdims=True)
    lse = m + jnp.log(jnp.sum(jnp.exp(z - m), axis=1, keepdims=True))
    out_ref[...] = z - lse


def _tc_layer1(agg1, deg, x_pad, W1l, b1, W1r, W2l, W2r):
    grid = (NPAD // BR,)
    return pl.pallas_call(
        _tc_layer1_body,
        grid=grid,
        in_specs=[
            pl.BlockSpec((NC, BR, CW1), lambda r: (0, r, 0)),
            pl.BlockSpec((NC, BR, 16), lambda r: (0, r, 0)),
            pl.BlockSpec((BR, NFEAT), lambda r: (r, 0)),
            pl.BlockSpec((NHID, NFEAT), lambda r: (0, 0)),
            pl.BlockSpec((1, NHID), lambda r: (0, 0)),
            pl.BlockSpec((NHID, NFEAT), lambda r: (0, 0)),
            pl.BlockSpec((NCLASS, NHID), lambda r: (0, 0)),
            pl.BlockSpec((NCLASS, NHID), lambda r: (0, 0)),
        ],
        out_specs=[
            pl.BlockSpec((NC, BR, CW2), lambda r: (0, r, 0)),
            pl.BlockSpec((BR, NCLASS), lambda r: (r, 0)),
        ],
        out_shape=[
            jax.ShapeDtypeStruct((NC, NPAD, CW2), jnp.float32),
            jax.ShapeDtypeStruct((NPAD, NCLASS), jnp.float32),
        ],
    )(agg1, deg, x_pad, W1l, b1, W1r, W2l, W2r)


def _tc_layer2(agg2, deg, hr, b2):
    grid = (NPAD // BR,)
    return pl.pallas_call(
        _tc_layer2_body,
        grid=grid,
        in_specs=[
            pl.BlockSpec((NC, BR, CW2), lambda r: (0, r, 0)),
            pl.BlockSpec((NC, BR, 16), lambda r: (0, r, 0)),
            pl.BlockSpec((BR, NCLASS), lambda r: (r, 0)),
            pl.BlockSpec((1, NCLASS), lambda r: (0, 0)),
        ],
        out_specs=pl.BlockSpec((BR, NCLASS), lambda r: (r, 0)),
        out_shape=jax.ShapeDtypeStruct((NPAD, NCLASS), jnp.float32),
    )(agg2, deg, hr, b2)


@jax.jit
def kernel(x, edge_index, W1l, b1, W1r, W2l, b2, W2r):
    src = edge_index[0]
    dst = edge_index[1]
    pad = E_PAD - E
    srcp = jnp.concatenate([src, jnp.zeros((pad,), jnp.int32)])
    dstp = jnp.concatenate([dst, jnp.full((pad,), DUMMY_DST, jnp.int32)])
    src3 = srcp.reshape(NS, CHUNKS_PER_TILE, CHUNK)
    src4 = jnp.stack([src3, src3 + NPAD])
    dst3 = dstp.reshape(NS, CHUNKS_PER_TILE, CHUNK)

    x_pad = jnp.pad(x, ((0, NPAD - N), (0, 0)))
    xcat = jnp.concatenate([x_pad[:, :CW1], x_pad[:, CW1:]], axis=0)

    agg1, deg = _sc_agg_l1(xcat, src4, dst3)

    hl, hr = _tc_layer1(agg1, deg, x_pad, W1l, b1.reshape(1, NHID), W1r,
                        W2l, W2r)

    (agg2,) = _sc_agg_l2(hl.reshape(NC * NPAD, CW2), src4, dst3)

    out = _tc_layer2(agg2, deg, hr, b2.reshape(1, NCLASS))
    return out[:N]
